# baseline (speedup 1.0000x reference)
import functools

import jax
import jax.numpy as jnp
from jax import lax
from jax.experimental import pallas as pl
from jax.experimental.pallas import tpu as pltpu
from jax.experimental.pallas import tpu_sc as plsc

_BATCH = 64
_NBEAM = 8
_VOCAB = 100000
_VMAIN = 99840
_VTAIL = _VOCAB - _VMAIN
_CELLV = 1664
_CPB = _VMAIN // _CELLV + 1
_NCELL = _NBEAM * _CPB
_NCPAD = 512
_CPW = 3
_WINV = _CPW * _CELLV
_NWIN = _VMAIN // _WINV
_TROW = _NBEAM * _VTAIL
_K = 16
_ROWS_PER_WORKER = 2
_NEG = float("-inf")
_BIGI = 2**31 - 1

_GDN = lax.GatherDimensionNumbers(
    offset_dims=(), collapsed_slice_dims=(0,), start_index_map=(0,))


def _shuf(x, lane, s):
    return lax.gather(x, (lane ^ s)[:, None], _GDN, (1,),
                      mode=lax.GatherScatterMode.PROMISE_IN_BOUNDS)


def _allred(x, lane, op):
    for s in (1, 2, 4, 8):
        x = op(x, _shuf(x, lane, s))
    return x


def _treemax(vals):
    while len(vals) > 1:
        vals = [jnp.maximum(vals[i], vals[i + 1])
                for i in range(0, len(vals) - 1, 2)] + (
                    [vals[-1]] if len(vals) % 2 else [])
    return vals[0]


def _topk_body(lp_hbm, tail_hbm, bias_hbm, vals_hbm, idx_hbm,
               win0, win1, cellbuf, tbuf, cellmax, stage_v, stage_j, biasv,
               sem0, sem1):
    wid = lax.axis_index("s") * 2 + lax.axis_index("c")
    lane = lax.iota(jnp.int32, 16)
    wins = (win0, win1)
    sems = (sem0, sem1)

    def row_body(rr, _unused):
        r = wid * _ROWS_PER_WORKER + rr
        pltpu.sync_copy(bias_hbm.at[r], biasv)
        pltpu.sync_copy(tail_hbm.at[pl.ds(r * _TROW, _TROW)], tbuf)
        bvec = biasv[...]
        bsel_s = [_allred(jnp.where(lane == s, bvec, _NEG), lane, jnp.maximum)
                  for s in range(_NBEAM)]

        neg16 = jnp.full((16,), _NEG, jnp.float32)
        cellmax[pl.ds(_NCPAD - 32, 16)] = neg16
        cellmax[pl.ds(_NCPAD - 16, 16)] = neg16

        def _start(g, par, r=r):
            pltpu.async_copy(lp_hbm.at[r, :, pl.ds(g * _WINV, _WINV)],
                             wins[par], sems[par])

        def _wait(g, par, r=r):
            pltpu.make_async_copy(lp_hbm.at[r, :, pl.ds(g * _WINV, _WINV)],
                                  wins[par], sems[par]).wait()

        def _compute(g, par):
            win = wins[par]
            for s in range(_NBEAM):
                for part in range(_CPW):
                    def i_body(i, acc, s=s, part=part):
                        base = part * _CELLV + i * 26 * 16
                        loads = [win[s, pl.ds(base + u * 16, 16)]
                                 for u in range(26)]
                        return jnp.maximum(acc, _treemax(loads))
                    acc = lax.fori_loop(0, 4, i_body,
                                        jnp.full((16,), _NEG, jnp.float32))
                    cm = _allred(acc, lane, jnp.maximum) + bsel_s[s]
                    cellid = s * _CPB + g * _CPW + part
                    gi = cellid // 16
                    li = cellid - gi * 16
                    cmv = cellmax[pl.ds(gi * 16, 16)]
                    cellmax[pl.ds(gi * 16, 16)] = jnp.where(lane == li, cm,
                                                            cmv)

        _start(0, 0)

        def p_body(p, _):
            for par in range(2):
                g = p * 2 + par

                _wait(g, par)

                @pl.when(g + 1 < _NWIN)
                def _():
                    _start(g + 1, (par + 1) % 2)

                _compute(g, par)
            return 0
        lax.fori_loop(0, _NWIN // 2, p_body, 0)

        for s in range(_NBEAM):
            acc = jnp.full((16,), _NEG, jnp.float32)
            for i in range(_VTAIL // 16):
                acc = jnp.maximum(acc, tbuf[pl.ds(s * _VTAIL + i * 16, 16)])
            cm = _allred(acc, lane, jnp.maximum) + bsel_s[s]
            cellid = s * _CPB + _CPB - 1
            gi = cellid // 16
            li = cellid - gi * 16
            cmv = cellmax[pl.ds(gi * 16, 16)]
            cellmax[pl.ds(gi * 16, 16)] = jnp.where(lane == li, cm, cmv)

        def t_body(t, carry, r=r, bvec=bvec):
            m_last, j_last, outv, outj = carry

            def a_body(a, macc):
                m1 = jnp.maximum(cellmax[pl.ds((a * 4 + 0) * 16, 16)],
                                 cellmax[pl.ds((a * 4 + 1) * 16, 16)])
                m2 = jnp.maximum(cellmax[pl.ds((a * 4 + 2) * 16, 16)],
                                 cellmax[pl.ds((a * 4 + 3) * 16, 16)])
                return jnp.maximum(macc, jnp.maximum(m1, m2))
            macc = lax.fori_loop(0, _NCPAD // 64, a_body,
                                 jnp.full((16,), _NEG, jnp.float32))
            m = _allred(macc, lane, jnp.maximum)

            def b_body(a, cacc):
                res = cacc
                for u in range(4):
                    gi = a * 4 + u
                    cmv = cellmax[pl.ds(gi * 16, 16)]
                    res = jnp.minimum(
                        res, jnp.where(cmv == m, gi * 16 + lane, _BIGI))
                return res
            cacc = lax.fori_loop(0, _NCPAD // 64, b_body,
                                 jnp.full((16,), _BIGI, jnp.int32))
            cs = _allred(cacc, lane, jnp.minimum)[0]
            beam = cs // _CPB
            k = cs - beam * _CPB
            bsel = _allred(jnp.where(lane == beam, bvec, _NEG), lane,
                           jnp.maximum)

            def scan_cell(read, nvreg, flat0, width):
                def de_body(i, acc3):
                    jacc, cnt, nlt = acc3
                    for u in range(width):
                        kk = i * width + u
                        v = read(kk) + bsel
                        flat = flat0 + kk * 16 + lane
                        cond = (v == m) & ((m < m_last) | (flat > j_last))
                        jacc = jnp.minimum(jacc,
                                           jnp.where(cond, flat, _BIGI))
                        cnt = cnt + jnp.where(cond, 1, 0)
                        nlt = jnp.maximum(nlt, jnp.where(v < m, v, _NEG))
                    return (jacc, cnt, nlt)
                return lax.fori_loop(
                    0, nvreg // width, de_body,
                    (jnp.full((16,), _BIGI, jnp.int32),
                     jnp.zeros((16,), jnp.int32),
                     jnp.full((16,), _NEG, jnp.float32)))

            kc = jnp.minimum(k, _CPB - 2)
            pltpu.sync_copy(
                lp_hbm.at[r, :, pl.ds(kc * _CELLV, _CELLV)], cellbuf)
            jm, cm_, nmlt = scan_cell(
                lambda kk: cellbuf[beam, pl.ds(kk * 16, 16)],
                _CELLV // 16, beam * _VOCAB + kc * _CELLV, 8)
            jt, ct, ntlt = scan_cell(
                lambda kk: tbuf[pl.ds(beam * _VTAIL + kk * 16, 16)],
                _VTAIL // 16, beam * _VOCAB + _VMAIN, 2)
            is_tail = k == _CPB - 1
            jacc = jnp.where(is_tail, jt, jm)
            cnt = jnp.where(is_tail, ct, cm_)
            nlt = jnp.where(is_tail, ntlt, nmlt)
            j = _allred(jacc, lane, jnp.minimum)
            cnts = _allred(cnt, lane, jnp.add)
            nm = jnp.where(cnts >= 2, m, _allred(nlt, lane, jnp.maximum))

            outv = jnp.where(lane == t, m, outv)
            outj = jnp.where(lane == t, j, outj)

            gi2 = cs // 16
            li2 = cs - gi2 * 16
            cmv2 = cellmax[pl.ds(gi2 * 16, 16)]
            cellmax[pl.ds(gi2 * 16, 16)] = jnp.where(lane == li2, nm, cmv2)
            return (m, j, outv, outj)

        init = (jnp.full((16,), float("inf"), jnp.float32),
                jnp.full((16,), -1, jnp.int32),
                jnp.full((16,), _NEG, jnp.float32),
                jnp.zeros((16,), jnp.int32))
        _, _, outv, outj = lax.fori_loop(0, _K, t_body, init)
        stage_v[...] = outv
        stage_j[...] = outj
        pltpu.sync_copy(stage_v, vals_hbm.at[r])
        pltpu.sync_copy(stage_j, idx_hbm.at[r])
        return 0

    lax.fori_loop(0, _ROWS_PER_WORKER, row_body, 0)


_sc_topk = functools.partial(
    pl.kernel,
    mesh=plsc.VectorSubcoreMesh(core_axis_name="c", subcore_axis_name="s"),
    out_type=[jax.ShapeDtypeStruct((_BATCH, _K), jnp.float32),
              jax.ShapeDtypeStruct((_BATCH, _K), jnp.int32)],
    scratch_types=[pltpu.VMEM((_NBEAM, _WINV), jnp.float32),
                   pltpu.VMEM((_NBEAM, _WINV), jnp.float32),
                   pltpu.VMEM((_NBEAM, _CELLV), jnp.float32),
                   pltpu.VMEM((_TROW,), jnp.float32),
                   pltpu.VMEM((_NCPAD,), jnp.float32),
                   pltpu.VMEM((_K,), jnp.float32),
                   pltpu.VMEM((_K,), jnp.int32),
                   pltpu.VMEM((16,), jnp.float32),
                   pltpu.SemaphoreType.DMA,
                   pltpu.SemaphoreType.DMA],
)(_topk_body)


def kernel(curr_lprobs, scores, step):
    batch, nbeam, vocab = curr_lprobs.shape
    step = jnp.asarray(step)
    beam_ids = jnp.arange(nbeam, dtype=jnp.int32)
    sel = scores[:, :, step - 1]
    bias = jnp.where(step == 0,
                     jnp.where(beam_ids[None, :] == 0, 0.0, -jnp.inf),
                     sel).astype(jnp.float32)
    bias16 = jnp.concatenate(
        [bias, jnp.zeros((batch, 16 - nbeam), jnp.float32)], axis=1)
    tail = curr_lprobs[:, :, _VMAIN:].reshape(batch * nbeam * _VTAIL)
    vals, fidx = _sc_topk(curr_lprobs, tail, bias16)
    beams = fidx // vocab
    idxs = fidx - beams * vocab
    return vals, idxs, beams


# speedup vs baseline: 1.8512x; 1.8512x over previous
import functools

import jax
import jax.numpy as jnp
from jax import lax
from jax.experimental import pallas as pl
from jax.experimental.pallas import tpu as pltpu
from jax.experimental.pallas import tpu_sc as plsc

_BATCH = 64
_NBEAM = 8
_VOCAB = 100000
_VMAIN = 99840
_VTAIL = _VOCAB - _VMAIN
_CELLV = 1664
_CPB = _VMAIN // _CELLV + 1
_NCELL = _NBEAM * _CPB
_NCPAD = 512
_WINV = 2 * _CELLV
_NWIN = _VMAIN // _WINV
_TROW = _NBEAM * _VTAIL
_K = 16
_ROWS_PER_WORKER = 2
_NEG = float("-inf")
_BIGI = 2**31 - 1

_GDN = lax.GatherDimensionNumbers(
    offset_dims=(), collapsed_slice_dims=(0,), start_index_map=(0,))


def _shuf(x, lane, s):
    return lax.gather(x, (lane ^ s)[:, None], _GDN, (1,),
                      mode=lax.GatherScatterMode.PROMISE_IN_BOUNDS)


def _allred(x, lane, op):
    for s in (1, 2, 4, 8):
        x = op(x, _shuf(x, lane, s))
    return x


def _topk_body(lp_hbm, tail_hbm, bias_hbm, vals_hbm, idx_hbm,
               win0, win1, cellbuf, tbuf, cellmax, stage_v, stage_j, biasv,
               sem0, sem1):
    wid = lax.axis_index("s") * 2 + lax.axis_index("c")
    lane = lax.iota(jnp.int32, 16)
    wins = (win0, win1)
    sems = (sem0, sem1)

    for rr in range(_ROWS_PER_WORKER):
        r = wid * _ROWS_PER_WORKER + rr
        pltpu.sync_copy(bias_hbm.at[r], biasv)
        pltpu.sync_copy(tail_hbm.at[pl.ds(r * _TROW, _TROW)], tbuf)
        bvec = biasv[...]
        bsel_s = [_allred(jnp.where(lane == s, bvec, _NEG), lane, jnp.maximum)
                  for s in range(_NBEAM)]

        neg16 = jnp.full((16,), _NEG, jnp.float32)
        cellmax[pl.ds(_NCPAD - 32, 16)] = neg16
        cellmax[pl.ds(_NCPAD - 16, 16)] = neg16

        def _start(g, par, r=r):
            pltpu.async_copy(lp_hbm.at[r, :, pl.ds(g * _WINV, _WINV)],
                             wins[par], sems[par])

        def _wait(g, par, r=r):
            pltpu.make_async_copy(lp_hbm.at[r, :, pl.ds(g * _WINV, _WINV)],
                                  wins[par], sems[par]).wait()

        def _compute(g, par):
            win = wins[par]
            for s in range(_NBEAM):
                for half in range(2):
                    def i_body(i, accs, s=s, half=half):
                        base = half * _CELLV + i * 8 * 16
                        new = []
                        for u in range(4):
                            t1 = jnp.maximum(
                                win[s, pl.ds(base + (u * 2 + 0) * 16, 16)],
                                win[s, pl.ds(base + (u * 2 + 1) * 16, 16)])
                            new.append(jnp.maximum(accs[u], t1))
                        return tuple(new)
                    accs = lax.fori_loop(
                        0, 13, i_body,
                        tuple(jnp.full((16,), _NEG, jnp.float32)
                              for _ in range(4)))
                    acc = jnp.maximum(jnp.maximum(accs[0], accs[1]),
                                      jnp.maximum(accs[2], accs[3]))
                    cm = _allred(acc, lane, jnp.maximum) + bsel_s[s]
                    cellid = s * _CPB + g * 2 + half
                    gi = cellid // 16
                    li = cellid - gi * 16
                    cmv = cellmax[pl.ds(gi * 16, 16)]
                    cellmax[pl.ds(gi * 16, 16)] = jnp.where(lane == li, cm,
                                                            cmv)

        _start(0, 0)

        def p_body(p, _):
            for par in range(2):
                g = p * 2 + par

                _wait(g, par)

                @pl.when(g + 1 < _NWIN)
                def _():
                    _start(g + 1, (par + 1) % 2)

                _compute(g, par)
            return 0
        lax.fori_loop(0, _NWIN // 2, p_body, 0)

        for s in range(_NBEAM):
            acc = jnp.full((16,), _NEG, jnp.float32)
            for i in range(_VTAIL // 16):
                acc = jnp.maximum(acc, tbuf[pl.ds(s * _VTAIL + i * 16, 16)])
            cm = _allred(acc, lane, jnp.maximum) + bsel_s[s]
            cellid = s * _CPB + _CPB - 1
            gi = cellid // 16
            li = cellid - gi * 16
            cmv = cellmax[pl.ds(gi * 16, 16)]
            cellmax[pl.ds(gi * 16, 16)] = jnp.where(lane == li, cm, cmv)

        def t_body(t, carry, r=r, bvec=bvec):
            m_last, j_last, outv, outj = carry

            def a_body(a, macc):
                m1 = jnp.maximum(cellmax[pl.ds((a * 4 + 0) * 16, 16)],
                                 cellmax[pl.ds((a * 4 + 1) * 16, 16)])
                m2 = jnp.maximum(cellmax[pl.ds((a * 4 + 2) * 16, 16)],
                                 cellmax[pl.ds((a * 4 + 3) * 16, 16)])
                return jnp.maximum(macc, jnp.maximum(m1, m2))
            macc = lax.fori_loop(0, _NCPAD // 64, a_body,
                                 jnp.full((16,), _NEG, jnp.float32))
            m = _allred(macc, lane, jnp.maximum)

            def b_body(a, cacc):
                res = cacc
                for u in range(4):
                    gi = a * 4 + u
                    cmv = cellmax[pl.ds(gi * 16, 16)]
                    res = jnp.minimum(
                        res, jnp.where(cmv == m, gi * 16 + lane, _BIGI))
                return res
            cacc = lax.fori_loop(0, _NCPAD // 64, b_body,
                                 jnp.full((16,), _BIGI, jnp.int32))
            cs = _allred(cacc, lane, jnp.minimum)[0]
            beam = cs // _CPB
            k = cs - beam * _CPB
            bsel = _allred(jnp.where(lane == beam, bvec, _NEG), lane,
                           jnp.maximum)

            def scan_cell(read, nvreg, flat0):
                def de_body(i, acc3):
                    jacc, cnt, nlt = acc3
                    for u in range(2):
                        kk = i * 2 + u
                        v = read(kk) + bsel
                        flat = flat0 + kk * 16 + lane
                        cond = (v == m) & ((m < m_last) | (flat > j_last))
                        jacc = jnp.minimum(jacc,
                                           jnp.where(cond, flat, _BIGI))
                        cnt = cnt + jnp.where(cond, 1, 0)
                        nlt = jnp.maximum(nlt, jnp.where(v < m, v, _NEG))
                    return (jacc, cnt, nlt)
                return lax.fori_loop(
                    0, nvreg // 2, de_body,
                    (jnp.full((16,), _BIGI, jnp.int32),
                     jnp.zeros((16,), jnp.int32),
                     jnp.full((16,), _NEG, jnp.float32)))

            kc = jnp.minimum(k, _CPB - 2)
            pltpu.sync_copy(
                lp_hbm.at[r, :, pl.ds(kc * _CELLV, _CELLV)], cellbuf)
            jm, cm_, nmlt = scan_cell(
                lambda kk: cellbuf[beam, pl.ds(kk * 16, 16)],
                _CELLV // 16, beam * _VOCAB + kc * _CELLV)
            jt, ct, ntlt = scan_cell(
                lambda kk: tbuf[pl.ds(beam * _VTAIL + kk * 16, 16)],
                _VTAIL // 16, beam * _VOCAB + _VMAIN)
            is_tail = k == _CPB - 1
            jacc = jnp.where(is_tail, jt, jm)
            cnt = jnp.where(is_tail, ct, cm_)
            nlt = jnp.where(is_tail, ntlt, nmlt)
            j = _allred(jacc, lane, jnp.minimum)
            cnts = _allred(cnt, lane, jnp.add)
            nm = jnp.where(cnts >= 2, m, _allred(nlt, lane, jnp.maximum))

            outv = jnp.where(lane == t, m, outv)
            outj = jnp.where(lane == t, j, outj)

            gi2 = cs // 16
            li2 = cs - gi2 * 16
            cmv2 = cellmax[pl.ds(gi2 * 16, 16)]
            cellmax[pl.ds(gi2 * 16, 16)] = jnp.where(lane == li2, nm, cmv2)
            return (m, j, outv, outj)

        init = (jnp.full((16,), float("inf"), jnp.float32),
                jnp.full((16,), -1, jnp.int32),
                jnp.full((16,), _NEG, jnp.float32),
                jnp.zeros((16,), jnp.int32))
        _, _, outv, outj = lax.fori_loop(0, _K, t_body, init)
        stage_v[...] = outv
        stage_j[...] = outj
        pltpu.sync_copy(stage_v, vals_hbm.at[r])
        pltpu.sync_copy(stage_j, idx_hbm.at[r])


_sc_topk = functools.partial(
    pl.kernel,
    mesh=plsc.VectorSubcoreMesh(core_axis_name="c", subcore_axis_name="s"),
    out_type=[jax.ShapeDtypeStruct((_BATCH, _K), jnp.float32),
              jax.ShapeDtypeStruct((_BATCH, _K), jnp.int32)],
    scratch_types=[pltpu.VMEM((_NBEAM, _WINV), jnp.float32),
                   pltpu.VMEM((_NBEAM, _WINV), jnp.float32),
                   pltpu.VMEM((_NBEAM, _CELLV), jnp.float32),
                   pltpu.VMEM((_TROW,), jnp.float32),
                   pltpu.VMEM((_NCPAD,), jnp.float32),
                   pltpu.VMEM((_K,), jnp.float32),
                   pltpu.VMEM((_K,), jnp.int32),
                   pltpu.VMEM((16,), jnp.float32),
                   pltpu.SemaphoreType.DMA,
                   pltpu.SemaphoreType.DMA],
)(_topk_body)


def kernel(curr_lprobs, scores, step):
    batch, nbeam, vocab = curr_lprobs.shape
    step = jnp.asarray(step)
    beam_ids = jnp.arange(nbeam, dtype=jnp.int32)
    sel = scores[:, :, step - 1]
    bias = jnp.where(step == 0,
                     jnp.where(beam_ids[None, :] == 0, 0.0, -jnp.inf),
                     sel).astype(jnp.float32)
    bias16 = jnp.concatenate(
        [bias, jnp.zeros((batch, 16 - nbeam), jnp.float32)], axis=1)
    tail = curr_lprobs[:, :, _VMAIN:].reshape(batch * nbeam * _VTAIL)
    vals, fidx = _sc_topk(curr_lprobs, tail, bias16)
    beams = fidx // vocab
    idxs = fidx - beams * vocab
    return vals, idxs, beams
